# Initial kernel scaffold; baseline (speedup 1.0000x reference)
import jax
import jax.numpy as jnp
from jax.experimental import pallas as pl


def kernel(v1, v2, v3, y, idx, memory_v1, memory_v2, memory_v3):
    raise NotImplementedError("write your pallas kernel here")


# speedup vs baseline: 5.9941x; 5.9941x over previous
import functools

import jax
import jax.numpy as jnp
from jax import lax
from jax.experimental import pallas as pl
from jax.experimental.pallas import tpu as pltpu
from jax.experimental.pallas import tpu_sc as plsc

_B = 64
_K1 = 4097
_D = 128
_N = 100000
_T = 0.07
_KP = 4112
_NROWS = 4 * _B
_NTILES = 32
_TASKS_PER_TILE = _NROWS // _NTILES
_CHUNK = 2048


def _tc_body(v12_ref, v3_ref, m1_ref, m2_ref, m3_ref, out_ref):
    dn = (((1,), (1,)), ((), ()))
    m3 = m3_ref[...]
    out_ref[0:128, :] = lax.dot_general(
        v12_ref[...], m3, dn, preferred_element_type=jnp.float32)
    out_ref[128:192, :] = lax.dot_general(
        v3_ref[...], m1_ref[...], dn, preferred_element_type=jnp.float32)
    out_ref[192:256, :] = lax.dot_general(
        v3_ref[...], m2_ref[...], dn, preferred_element_type=jnp.float32)


def _tc_logits(v12, v3, m1, m2, m3):
    return pl.pallas_call(
        _tc_body,
        grid=(pl.cdiv(_N, _CHUNK),),
        in_specs=[
            pl.BlockSpec((2 * _B, _D), lambda i: (0, 0)),
            pl.BlockSpec((_B, _D), lambda i: (0, 0)),
            pl.BlockSpec((_CHUNK, _D), lambda i: (i, 0)),
            pl.BlockSpec((_CHUNK, _D), lambda i: (i, 0)),
            pl.BlockSpec((_CHUNK, _D), lambda i: (i, 0)),
        ],
        out_specs=pl.BlockSpec((_NROWS, _CHUNK), lambda i: (0, i)),
        out_shape=jax.ShapeDtypeStruct((_NROWS, _N), jnp.float32),
    )(v12, v3, m1, m2, m3)


_sc_mesh = plsc.VectorSubcoreMesh(core_axis_name="c", subcore_axis_name="s")


@functools.partial(
    pl.kernel,
    out_type=jax.ShapeDtypeStruct((_NROWS * _KP,), jnp.float32),
    mesh=_sc_mesh,
    compiler_params=pltpu.CompilerParams(needs_layout_passes=False),
    scratch_types=[
        pltpu.VMEM((_N,), jnp.float32),
        pltpu.VMEM((_KP,), jnp.int32),
        pltpu.VMEM((_KP,), jnp.float32),
    ],
)
def _sc_gather(p_hbm, idx_hbm, out_hbm, prow_v, idx_v, out_v):
    wid = lax.axis_index("s") * 2 + lax.axis_index("c")
    inv_t = jnp.float32(1.0 / _T)

    def task(t, carry):
        r = wid * _TASKS_PER_TILE + t
        b = lax.rem(r, _B)
        pltpu.sync_copy(p_hbm.at[pl.ds(pl.multiple_of(r * _N, 8), _N)], prow_v)
        pltpu.sync_copy(
            idx_hbm.at[pl.ds(pl.multiple_of(b * _KP, 8), _KP)], idx_v)

        def col(i, c):
            sl = pl.ds(i * 16, 16)
            vals = plsc.load_gather(prow_v, [idx_v[sl]])
            out_v[sl] = vals * inv_t
            return c

        lax.fori_loop(0, _KP // 16, col, 0)
        pltpu.sync_copy(
            out_v, out_hbm.at[pl.ds(pl.multiple_of(r * _KP, 8), _KP)])
        return carry

    lax.fori_loop(0, _TASKS_PER_TILE, task, 0)


def kernel(v1, v2, v3, y, idx, memory_v1, memory_v2, memory_v3):
    del y
    v12 = jnp.concatenate([v1, v2], axis=0)
    logits = _tc_logits(v12, v3, memory_v1, memory_v2, memory_v3)
    idx_p = jnp.pad(idx, ((0, 0), (0, _KP - _K1))).reshape(-1)
    flat = _sc_gather(logits.reshape(-1), idx_p)
    o = flat.reshape(4, _B, _KP)[:, :, :_K1, None]
    return (o[0], o[1], o[2], o[3])


# speedup vs baseline: 6.2175x; 1.0373x over previous
import functools

import jax
import jax.numpy as jnp
from jax import lax
from jax.experimental import pallas as pl
from jax.experimental.pallas import tpu as pltpu
from jax.experimental.pallas import tpu_sc as plsc

_B = 64
_K1 = 4097
_D = 128
_N = 100000
_T = 0.07
_KP = 4112
_NROWS = 4 * _B
_NTILES = 32
_TASKS_PER_TILE = _NROWS // _NTILES
_CHUNK = 8192


def _tc_body(v12_ref, v3_ref, m1_ref, m2_ref, m3_ref, out_ref):
    dn = (((1,), (1,)), ((), ()))
    m3 = m3_ref[...]
    out_ref[0:128, :] = lax.dot_general(
        v12_ref[...], m3, dn, preferred_element_type=jnp.float32)
    out_ref[128:192, :] = lax.dot_general(
        v3_ref[...], m1_ref[...], dn, preferred_element_type=jnp.float32)
    out_ref[192:256, :] = lax.dot_general(
        v3_ref[...], m2_ref[...], dn, preferred_element_type=jnp.float32)


def _tc_logits(v12, v3, m1, m2, m3):
    return pl.pallas_call(
        _tc_body,
        grid=(pl.cdiv(_N, _CHUNK),),
        in_specs=[
            pl.BlockSpec((2 * _B, _D), lambda i: (0, 0)),
            pl.BlockSpec((_B, _D), lambda i: (0, 0)),
            pl.BlockSpec((_CHUNK, _D), lambda i: (i, 0)),
            pl.BlockSpec((_CHUNK, _D), lambda i: (i, 0)),
            pl.BlockSpec((_CHUNK, _D), lambda i: (i, 0)),
        ],
        out_specs=pl.BlockSpec((_NROWS, _CHUNK), lambda i: (0, i)),
        out_shape=jax.ShapeDtypeStruct((_NROWS, _N), jnp.float32),
    )(v12, v3, m1, m2, m3)


_sc_mesh = plsc.VectorSubcoreMesh(core_axis_name="c", subcore_axis_name="s")


@functools.partial(
    pl.kernel,
    out_type=jax.ShapeDtypeStruct((_NROWS * _KP,), jnp.float32),
    mesh=_sc_mesh,
    compiler_params=pltpu.CompilerParams(needs_layout_passes=False),
    scratch_types=[
        pltpu.VMEM((_N,), jnp.float32),
        pltpu.VMEM((_KP,), jnp.int32),
        pltpu.VMEM((_KP,), jnp.float32),
    ],
)
def _sc_gather(p_hbm, idx_hbm, out_hbm, prow_v, idx_v, out_v):
    wid = lax.axis_index("s") * 2 + lax.axis_index("c")
    inv_t = jnp.float32(1.0 / _T)

    def task(t, carry):
        r = wid * _TASKS_PER_TILE + t
        b = lax.rem(r, _B)
        pltpu.sync_copy(p_hbm.at[pl.ds(pl.multiple_of(r * _N, 8), _N)], prow_v)
        pltpu.sync_copy(
            idx_hbm.at[pl.ds(pl.multiple_of(b * _KP, 8), _KP)], idx_v)

        def col(i, c):
            sl = pl.ds(i * 16, 16)
            vals = plsc.load_gather(prow_v, [idx_v[sl]])
            out_v[sl] = vals * inv_t
            return c

        lax.fori_loop(0, _KP // 16, col, 0)
        pltpu.sync_copy(
            out_v, out_hbm.at[pl.ds(pl.multiple_of(r * _KP, 8), _KP)])
        return carry

    lax.fori_loop(0, _TASKS_PER_TILE, task, 0)


def kernel(v1, v2, v3, y, idx, memory_v1, memory_v2, memory_v3):
    del y
    v12 = jnp.concatenate([v1, v2], axis=0)
    logits = _tc_logits(v12, v3, memory_v1, memory_v2, memory_v3)
    idx_p = jnp.pad(idx, ((0, 0), (0, _KP - _K1))).reshape(-1)
    flat = _sc_gather(logits.reshape(-1), idx_p)
    o = flat.reshape(4, _B, _KP)[:, :, :_K1, None]
    return (o[0], o[1], o[2], o[3])


# speedup vs baseline: 9.3570x; 1.5049x over previous
import functools

import jax
import jax.numpy as jnp
from jax import lax
from jax.experimental import pallas as pl
from jax.experimental.pallas import tpu as pltpu
from jax.experimental.pallas import tpu_sc as plsc

_B = 64
_K1 = 4097
_D = 128
_N = 100000
_T = 0.07
_KP = 4112
_NROWS = 4 * _B
_NPACK = _NROWS // 2
_NTILES = 32
_TASKS_PER_TILE = _NPACK // _NTILES
_CHUNK = 8192


def _tc_body(v12_ref, v3_ref, m1_ref, m2_ref, m3_ref, out_ref):
    dn = (((1,), (1,)), ((), ()))

    def packed(lhs, rhs):
        prod = lax.dot_general(lhs, rhs, dn,
                               preferred_element_type=jnp.float32)
        return pltpu.bitcast(prod.astype(jnp.bfloat16), jnp.int32)

    out_ref[0:64, :] = packed(v12_ref[...], m3_ref[...])
    out_ref[64:96, :] = packed(v3_ref[...], m1_ref[...])
    out_ref[96:128, :] = packed(v3_ref[...], m2_ref[...])


def _tc_logits(v12, v3, m1, m2, m3):
    return pl.pallas_call(
        _tc_body,
        grid=(pl.cdiv(_N, _CHUNK),),
        in_specs=[
            pl.BlockSpec((2 * _B, _D), lambda i: (0, 0)),
            pl.BlockSpec((_B, _D), lambda i: (0, 0)),
            pl.BlockSpec((_CHUNK, _D), lambda i: (i, 0)),
            pl.BlockSpec((_CHUNK, _D), lambda i: (i, 0)),
            pl.BlockSpec((_CHUNK, _D), lambda i: (i, 0)),
        ],
        out_specs=pl.BlockSpec((_NPACK, _CHUNK), lambda i: (0, i)),
        out_shape=jax.ShapeDtypeStruct((_NPACK, _N), jnp.int32),
    )(v12, v3, m1, m2, m3)


_sc_mesh = plsc.VectorSubcoreMesh(core_axis_name="c", subcore_axis_name="s")


@functools.partial(
    pl.kernel,
    out_type=jax.ShapeDtypeStruct((_NROWS * _KP,), jnp.float32),
    mesh=_sc_mesh,
    compiler_params=pltpu.CompilerParams(needs_layout_passes=False),
    scratch_types=[
        pltpu.VMEM((_N,), jnp.int32),
        pltpu.VMEM((_KP,), jnp.int32),
        pltpu.VMEM((_KP,), jnp.int32),
        pltpu.VMEM((_KP,), jnp.float32),
        pltpu.VMEM((_KP,), jnp.float32),
    ],
)
def _sc_gather(p_hbm, idx_hbm, out_hbm, prow_v, idx0_v, idx1_v,
               out0_v, out1_v):
    wid = lax.axis_index("s") * 2 + lax.axis_index("c")
    inv_t = jnp.float32(1.0 / _T)
    hi_mask = jnp.int32(-65536)

    def task(t, carry):
        m = wid * _TASKS_PER_TILE + t
        r0 = 2 * m
        b0 = lax.rem(r0, _B)
        b1 = lax.rem(r0 + 1, _B)
        pltpu.sync_copy(p_hbm.at[pl.ds(pl.multiple_of(m * _N, 8), _N)], prow_v)
        pltpu.sync_copy(
            idx_hbm.at[pl.ds(pl.multiple_of(b0 * _KP, 8), _KP)], idx0_v)
        pltpu.sync_copy(
            idx_hbm.at[pl.ds(pl.multiple_of(b1 * _KP, 8), _KP)], idx1_v)

        def col(i, c):
            sl = pl.ds(i * 16, 16)
            w0 = plsc.load_gather(prow_v, [idx0_v[sl]])
            out0_v[sl] = plsc.bitcast(
                lax.shift_left(w0, 16), jnp.float32) * inv_t
            w1 = plsc.load_gather(prow_v, [idx1_v[sl]])
            out1_v[sl] = plsc.bitcast(
                lax.bitwise_and(w1, hi_mask), jnp.float32) * inv_t
            return c

        lax.fori_loop(0, _KP // 16, col, 0)
        pltpu.sync_copy(
            out0_v, out_hbm.at[pl.ds(pl.multiple_of(r0 * _KP, 8), _KP)])
        pltpu.sync_copy(
            out1_v,
            out_hbm.at[pl.ds(pl.multiple_of((r0 + 1) * _KP, 8), _KP)])
        return carry

    lax.fori_loop(0, _TASKS_PER_TILE, task, 0)


def kernel(v1, v2, v3, y, idx, memory_v1, memory_v2, memory_v3):
    del y
    v12 = jnp.concatenate([v1, v2], axis=0)
    logits = _tc_logits(v12, v3, memory_v1, memory_v2, memory_v3)
    idx_p = jnp.pad(idx, ((0, 0), (0, _KP - _K1))).reshape(-1)
    flat = _sc_gather(logits.reshape(-1), idx_p)
    o = flat.reshape(4, _B, _KP)[:, :, :_K1, None]
    return (o[0], o[1], o[2], o[3])


# speedup vs baseline: 9.6256x; 1.0287x over previous
import functools

import jax
import jax.numpy as jnp
from jax import lax
from jax.experimental import pallas as pl
from jax.experimental.pallas import tpu as pltpu
from jax.experimental.pallas import tpu_sc as plsc

_B = 64
_K1 = 4097
_D = 128
_N = 100000
_T = 0.07
_KP = 4112
_NPACK_H = 64
_NTILES = 32
_TASKS_PER_TILE = _NPACK_H // _NTILES
_CHUNK = 8192


def _packed(lhs, rhs):
    dn = (((1,), (1,)), ((), ()))
    prod = lax.dot_general(lhs, rhs, dn, preferred_element_type=jnp.float32)
    return pltpu.bitcast(prod.astype(jnp.bfloat16), jnp.int32)


def _tc_body_a(v12_ref, m3_ref, out_ref):
    out_ref[...] = _packed(v12_ref[...], m3_ref[...])


def _tc_body_b(v3_ref, m1_ref, m2_ref, out_ref):
    out_ref[0:32, :] = _packed(v3_ref[...], m1_ref[...])
    out_ref[32:64, :] = _packed(v3_ref[...], m2_ref[...])


def _tc_call(body, vecs, banks):
    full = lambda shape: pl.BlockSpec(shape, lambda i: (0, 0))
    chunk = pl.BlockSpec((_CHUNK, _D), lambda i: (i, 0))
    return pl.pallas_call(
        body,
        grid=(pl.cdiv(_N, _CHUNK),),
        in_specs=[full(v.shape) for v in vecs] + [chunk] * len(banks),
        out_specs=pl.BlockSpec((_NPACK_H, _CHUNK), lambda i: (0, i)),
        out_shape=jax.ShapeDtypeStruct((_NPACK_H, _N), jnp.int32),
    )(*vecs, *banks)


_sc_mesh = plsc.VectorSubcoreMesh(core_axis_name="c", subcore_axis_name="s")


@functools.partial(
    pl.kernel,
    out_type=jax.ShapeDtypeStruct((2 * _NPACK_H * _KP,), jnp.float32),
    mesh=_sc_mesh,
    compiler_params=pltpu.CompilerParams(needs_layout_passes=False),
    scratch_types=[
        pltpu.VMEM((_N,), jnp.int32),
        pltpu.VMEM((_KP,), jnp.int32),
        pltpu.VMEM((_KP,), jnp.int32),
        pltpu.VMEM((_KP,), jnp.float32),
        pltpu.VMEM((_KP,), jnp.float32),
    ],
)
def _sc_gather(p_hbm, idx_hbm, out_hbm, prow_v, idx0_v, idx1_v,
               out0_v, out1_v):
    wid = lax.axis_index("s") * 2 + lax.axis_index("c")
    inv_t = jnp.float32(1.0 / _T)
    hi_mask = jnp.int32(-65536)

    def task(t, carry):
        m = wid * _TASKS_PER_TILE + t
        r0 = 2 * m
        b0 = lax.rem(r0, _B)
        b1 = lax.rem(r0 + 1, _B)
        pltpu.sync_copy(p_hbm.at[pl.ds(pl.multiple_of(m * _N, 8), _N)], prow_v)
        pltpu.sync_copy(
            idx_hbm.at[pl.ds(pl.multiple_of(b0 * _KP, 8), _KP)], idx0_v)
        pltpu.sync_copy(
            idx_hbm.at[pl.ds(pl.multiple_of(b1 * _KP, 8), _KP)], idx1_v)

        def col(i, c):
            sl = pl.ds(i * 16, 16)
            w0 = plsc.load_gather(prow_v, [idx0_v[sl]])
            out0_v[sl] = plsc.bitcast(
                lax.shift_left(w0, 16), jnp.float32) * inv_t
            w1 = plsc.load_gather(prow_v, [idx1_v[sl]])
            out1_v[sl] = plsc.bitcast(
                lax.bitwise_and(w1, hi_mask), jnp.float32) * inv_t
            return c

        lax.fori_loop(0, _KP // 16, col, 0)
        pltpu.sync_copy(
            out0_v, out_hbm.at[pl.ds(pl.multiple_of(r0 * _KP, 8), _KP)])
        pltpu.sync_copy(
            out1_v,
            out_hbm.at[pl.ds(pl.multiple_of((r0 + 1) * _KP, 8), _KP)])
        return carry

    lax.fori_loop(0, _TASKS_PER_TILE, task, 0)


def kernel(v1, v2, v3, y, idx, memory_v1, memory_v2, memory_v3):
    del y
    v12 = jnp.concatenate([v1, v2], axis=0)
    pa = _tc_call(_tc_body_a, [v12], [memory_v3])
    pb = _tc_call(_tc_body_b, [v3], [memory_v1, memory_v2])
    idx_p = jnp.pad(idx, ((0, 0), (0, _KP - _K1))).reshape(-1)
    fa = _sc_gather(pa.reshape(-1), idx_p)
    fb = _sc_gather(pb.reshape(-1), idx_p)
    o = jnp.concatenate([fa, fb]).reshape(4, _B, _KP)[:, :, :_K1, None]
    return (o[0], o[1], o[2], o[3])


# speedup vs baseline: 14.5438x; 1.5110x over previous
import functools

import jax
import jax.numpy as jnp
from jax import lax
from jax.experimental import pallas as pl
from jax.experimental.pallas import tpu as pltpu
from jax.experimental.pallas import tpu_sc as plsc

_B = 64
_K1 = 4097
_D = 128
_N = 100000
_T = 0.07
_KP = 4112
_NPACK_H = 64
_NTILES = 32
_TASKS_PER_TILE = _NPACK_H // _NTILES
_CHUNK = 8192
_NBLK = 13
_NP = _NBLK * _CHUNK


def _packed(lhs, rhs):
    dn = (((1,), (1,)), ((), ()))
    prod = lax.dot_general(lhs, rhs, dn, preferred_element_type=jnp.float32)
    return pltpu.bitcast(prod.astype(jnp.bfloat16), jnp.int32)


def _tc_body_a(v12_ref, m3_ref, out_ref):
    out_ref[...] = _packed(v12_ref[...], m3_ref[...]).reshape(
        _NPACK_H, _CHUNK // 128, 128)


def _tc_body_b(v3_ref, m1_ref, m2_ref, out_ref):
    out_ref[0:32] = _packed(v3_ref[...], m1_ref[...]).reshape(
        32, _CHUNK // 128, 128)
    out_ref[32:64] = _packed(v3_ref[...], m2_ref[...]).reshape(
        32, _CHUNK // 128, 128)


def _tc_call(body, vecs, banks):
    full = lambda shape: pl.BlockSpec(shape, lambda i: (0, 0))
    chunk = pl.BlockSpec((_CHUNK, _D), lambda i: (i, 0))
    return pl.pallas_call(
        body,
        grid=(_NBLK,),
        in_specs=[full(v.shape) for v in vecs] + [chunk] * len(banks),
        out_specs=pl.BlockSpec((_NPACK_H, _CHUNK // 128, 128),
                               lambda i: (0, i, 0)),
        out_shape=jax.ShapeDtypeStruct((_NPACK_H, _NP // 128, 128),
                                       jnp.int32),
    )(*vecs, *banks)


_sc_mesh = plsc.VectorSubcoreMesh(core_axis_name="c", subcore_axis_name="s")


@functools.partial(
    pl.kernel,
    out_type=jax.ShapeDtypeStruct((2 * _NPACK_H * _KP,), jnp.float32),
    mesh=_sc_mesh,
    compiler_params=pltpu.CompilerParams(needs_layout_passes=False),
    scratch_types=[
        pltpu.VMEM((_NP,), jnp.int32),
        pltpu.VMEM((_KP,), jnp.int32),
        pltpu.VMEM((_KP,), jnp.int32),
        pltpu.VMEM((_KP,), jnp.float32),
        pltpu.VMEM((_KP,), jnp.float32),
    ],
)
def _sc_gather(p_hbm, idx_hbm, out_hbm, prow_v, idx0_v, idx1_v,
               out0_v, out1_v):
    wid = lax.axis_index("s") * 2 + lax.axis_index("c")
    inv_t = jnp.float32(1.0 / _T)
    hi_mask = jnp.int32(-65536)

    def task(t, carry):
        m = wid * _TASKS_PER_TILE + t
        r0 = 2 * m
        b0 = lax.rem(r0, _B)
        b1 = lax.rem(r0 + 1, _B)
        pltpu.sync_copy(p_hbm.at[pl.ds(pl.multiple_of(m * _NP, 8), _NP)], prow_v)
        pltpu.sync_copy(
            idx_hbm.at[pl.ds(pl.multiple_of(b0 * _KP, 8), _KP)], idx0_v)
        pltpu.sync_copy(
            idx_hbm.at[pl.ds(pl.multiple_of(b1 * _KP, 8), _KP)], idx1_v)

        def col(i, c):
            sl = pl.ds(i * 16, 16)
            w0 = plsc.load_gather(prow_v, [idx0_v[sl]])
            out0_v[sl] = plsc.bitcast(
                lax.shift_left(w0, 16), jnp.float32) * inv_t
            w1 = plsc.load_gather(prow_v, [idx1_v[sl]])
            out1_v[sl] = plsc.bitcast(
                lax.bitwise_and(w1, hi_mask), jnp.float32) * inv_t
            return c

        lax.fori_loop(0, _KP // 16, col, 0)
        pltpu.sync_copy(
            out0_v, out_hbm.at[pl.ds(pl.multiple_of(r0 * _KP, 8), _KP)])
        pltpu.sync_copy(
            out1_v,
            out_hbm.at[pl.ds(pl.multiple_of((r0 + 1) * _KP, 8), _KP)])
        return carry

    lax.fori_loop(0, _TASKS_PER_TILE, task, 0)


def kernel(v1, v2, v3, y, idx, memory_v1, memory_v2, memory_v3):
    del y
    v12 = jnp.concatenate([v1, v2], axis=0)
    pa = _tc_call(_tc_body_a, [v12], [memory_v3])
    pb = _tc_call(_tc_body_b, [v3], [memory_v1, memory_v2])
    idx_p = jnp.pad(idx, ((0, 0), (0, _KP - _K1))).reshape(-1)
    fa = _sc_gather(pa.reshape(-1), idx_p)
    fb = _sc_gather(pb.reshape(-1), idx_p)
    o = jnp.concatenate([fa, fb]).reshape(4, _B, _KP)[:, :, :_K1, None]
    return (o[0], o[1], o[2], o[3])
